# Initial kernel scaffold; baseline (speedup 1.0000x reference)
import functools

import jax
import jax.numpy as jnp
from jax import lax
from jax.experimental import pallas as pl
from jax.experimental.pallas import tpu as pltpu
from jax.experimental.pallas import tpu_sc as plsc

N = 10000
E = 320000
T = 12
D = 128
H = 32
P = 12
OUT = 1

NC = 2
NS = 16
NW = NC * NS

NP = 10112
RPT = NP // NS
CHUNK = 128
CPT = 79
EPT = CPT * CHUNK
EP = NW * EPT


def _make_agg(w):
    mesh = plsc.VectorSubcoreMesh(core_axis_name="c", subcore_axis_name="s")

    @functools.partial(
        pl.kernel,
        out_type=jax.ShapeDtypeStruct((NC, NP, w), jnp.float32),
        mesh=mesh,
        scratch_types=[
            pltpu.VMEM((CPT, CHUNK), jnp.int32),
            pltpu.VMEM((CPT, CHUNK), jnp.int32),
            pltpu.VMEM((CHUNK, w), jnp.float32),
            pltpu.VMEM_SHARED((NP, w), jnp.float32),
            pltpu.SemaphoreType.DMA,
        ],
    )
    def agg(v_hbm, srcw_hbm, dstw_hbm, out_hbm, src_v, dst_v, buf, acc_sh, sem):
        c = lax.axis_index("c")
        s = lax.axis_index("s")
        wid = c * NS + s

        def zbody(i, carry):
            for hh in range(w // 16):
                buf[i, hh * 16:(hh + 1) * 16] = jnp.zeros((16,), jnp.float32)
            return carry

        lax.fori_loop(0, CHUNK, zbody, 0)
        row0 = s * RPT
        off = 0
        for sz in (128, 128, 128, 128, 120):
            pltpu.sync_copy(buf.at[pl.ds(0, sz)],
                            acc_sh.at[pl.ds(row0 + off, sz)])
            off += sz

        pltpu.sync_copy(srcw_hbm.at[wid], src_v)
        pltpu.sync_copy(dstw_hbm.at[wid], dst_v)
        plsc.subcore_barrier()

        def body(j, carry):
            pltpu.async_copy(v_hbm.at[src_v.at[j]], buf, sem).wait()
            pltpu.sync_copy(buf, acc_sh.at[dst_v.at[j]], add=True)
            return carry

        lax.fori_loop(0, CPT, body, 0)
        plsc.subcore_barrier()
        pltpu.sync_copy(acc_sh.at[pl.ds(row0, RPT)],
                        out_hbm.at[c, pl.ds(row0, RPT)])

    return agg


_agg32 = _make_agg(H)
_agg96 = _make_agg(3 * H)


def _xproj_kernel(x_ref, w_ref, o_ref):
    o_ref[0] = jnp.dot(x_ref[0], w_ref[...],
                       preferred_element_type=jnp.float32)


def _xterm1_kernel(xp_ref, ag_ref, o_ref):
    o_ref[0] = xp_ref[0] + ag_ref[0, 0] + ag_ref[0, 1]


def _xterm2_kernel(o1_ref, ag_ref, w_ref, o_ref):
    xa = o1_ref[0] + ag_ref[0, 0] + ag_ref[0, 1]
    o_ref[0] = jnp.dot(xa, w_ref[...], preferred_element_type=jnp.float32)


def _gate_a_kernel(xterm_ref, h_ref, aggh_ref, whzr_ref, bzr_ref,
                   z_ref, g_ref, xc_ref):
    h = h_ref[...]
    ha = h + aggh_ref[0] + aggh_ref[1]
    hz = jnp.dot(ha, whzr_ref[...],
                 preferred_element_type=jnp.float32) + bzr_ref[...]
    xt = xterm_ref[...]
    z = jax.nn.sigmoid(xt[:, :H] + hz[:, :H])
    r = jax.nn.sigmoid(xt[:, H:2 * H] + hz[:, H:])
    z_ref[...] = z
    g_ref[...] = r * h
    xc_ref[...] = xt[:, 2 * H:]


def _gate_b_kernel(z_ref, h_ref, g_ref, xc_ref, aggg_ref, whh_ref, bh_ref,
                   hn_ref):
    ga = g_ref[...] + aggg_ref[0] + aggg_ref[1]
    c = jnp.tanh(xc_ref[...] +
                 jnp.dot(ga, whh_ref[...], preferred_element_type=jnp.float32)
                 + bh_ref[...])
    z = z_ref[...]
    hn_ref[...] = z * h_ref[...] + (1.0 - z) * c


def _head_kernel(h_ref, w_ref, b_ref, o_ref):
    o_ref[...] = jax.nn.relu(
        jnp.dot(h_ref[...], w_ref[...], preferred_element_type=jnp.float32)
        + b_ref[...])


def _tc(body, out_shape, grid=None, in_specs=None, out_specs=None):
    kwargs = {}
    if grid is not None:
        kwargs = dict(grid=grid, in_specs=in_specs, out_specs=out_specs)
    return pl.pallas_call(body, out_shape=out_shape, **kwargs)


def kernel(x, edge_index, params):
    p1, p2 = params['l1'], params['l2']

    Wx1 = jnp.concatenate([p1['Wxz'], p1['Wxr'], p1['Wxh']], axis=1)
    Wx2 = jnp.concatenate([p2['Wxz'], p2['Wxr'], p2['Wxh']], axis=1)
    Whzr1 = jnp.concatenate([p1['Whz'], p1['Whr']], axis=1)
    Whzr2 = jnp.concatenate([p2['Whz'], p2['Whr']], axis=1)
    bzr1 = jnp.concatenate([p1['bz'], p1['br']]).reshape(1, 2 * H)
    bzr2 = jnp.concatenate([p2['bz'], p2['br']]).reshape(1, 2 * H)
    bh1 = p1['bh'].reshape(1, H)
    bh2 = p2['bh'].reshape(1, H)
    bfc = params['bfc'].reshape(1, P * OUT)

    pad = EP - E
    srcp = jnp.concatenate([edge_index[0],
                            jnp.zeros((pad,), jnp.int32)])
    dstp = jnp.concatenate([edge_index[1],
                            jnp.full((pad,), NP - 1, jnp.int32)])
    srcw = srcp.reshape(NW, CPT, CHUNK)
    dstw = dstp.reshape(NW, CPT, CHUNK)

    x_pad = jnp.pad(x, ((0, 0), (0, NP - N), (0, 0)))

    xp_all = _tc(
        _xproj_kernel,
        jax.ShapeDtypeStruct((T, NP, 3 * H), jnp.float32),
        grid=(T,),
        in_specs=[pl.BlockSpec((1, NP, D), lambda t: (t, 0, 0)),
                  pl.BlockSpec((D, 3 * H), lambda t: (0, 0))],
        out_specs=pl.BlockSpec((1, NP, 3 * H), lambda t: (t, 0, 0)),
    )(x_pad, Wx1)

    aggxp = lax.map(lambda v: _agg96(v, srcw, dstw), xp_all)

    xterm1 = _tc(
        _xterm1_kernel,
        jax.ShapeDtypeStruct((T, NP, 3 * H), jnp.float32),
        grid=(T,),
        in_specs=[pl.BlockSpec((1, NP, 3 * H), lambda t: (t, 0, 0)),
                  pl.BlockSpec((1, NC, NP, 3 * H), lambda t: (t, 0, 0, 0))],
        out_specs=pl.BlockSpec((1, NP, 3 * H), lambda t: (t, 0, 0)),
    )(xp_all, aggxp)

    def make_step(Whzr, bzr, Whh, bh, collect):
        def step(h, xterm_t):
            aggh = _agg32(h, srcw, dstw)
            z, g, xc = _tc(
                _gate_a_kernel,
                (jax.ShapeDtypeStruct((NP, H), jnp.float32),
                 jax.ShapeDtypeStruct((NP, H), jnp.float32),
                 jax.ShapeDtypeStruct((NP, H), jnp.float32)),
            )(xterm_t, h, aggh, Whzr, bzr)
            aggg = _agg32(g, srcw, dstw)
            hn = _tc(
                _gate_b_kernel,
                jax.ShapeDtypeStruct((NP, H), jnp.float32),
            )(z, h, g, xc, aggg, Whh, bh)
            return hn, (hn if collect else None)
        return step

    h0 = jnp.zeros((NP, H), jnp.float32)
    h1, out1 = lax.scan(make_step(Whzr1, bzr1, p1['Whh'], bh1, True),
                        h0, xterm1)

    aggo = lax.map(lambda v: _agg32(v, srcw, dstw), out1)
    xterm2 = _tc(
        _xterm2_kernel,
        jax.ShapeDtypeStruct((T, NP, 3 * H), jnp.float32),
        grid=(T,),
        in_specs=[pl.BlockSpec((1, NP, H), lambda t: (t, 0, 0)),
                  pl.BlockSpec((1, NC, NP, H), lambda t: (t, 0, 0, 0)),
                  pl.BlockSpec((H, 3 * H), lambda t: (0, 0))],
        out_specs=pl.BlockSpec((1, NP, 3 * H), lambda t: (t, 0, 0)),
    )(out1, aggo, Wx2)

    h2, _ = lax.scan(make_step(Whzr2, bzr2, p2['Whh'], bh2, False),
                     h1, xterm2)

    y = _tc(
        _head_kernel,
        jax.ShapeDtypeStruct((NP, P * OUT), jnp.float32),
    )(h2, params['Wfc'], bfc)

    y = y[:N].reshape(N, P, OUT)
    return jnp.swapaxes(y, 0, 1)


# speedup vs baseline: 7.1958x; 7.1958x over previous
import functools

import jax
import jax.numpy as jnp
from jax import lax
from jax.experimental import pallas as pl
from jax.experimental.pallas import tpu as pltpu
from jax.experimental.pallas import tpu_sc as plsc

N = 10000
E = 320000
T = 12
D = 128
H = 32
P = 12
OUT = 1

NC = 2
NS = 16
NW = NC * NS

NP = 10112
RPT = NP // NS
CHUNK = 128
CPT = 79
EPT = CPT * CHUNK
EP = NW * EPT


def _make_agg(w):
    mesh = plsc.VectorSubcoreMesh(core_axis_name="c", subcore_axis_name="s")

    @functools.partial(
        pl.kernel,
        out_type=jax.ShapeDtypeStruct((NC, NP, w), jnp.float32),
        mesh=mesh,
        scratch_types=[
            pltpu.VMEM((CPT, CHUNK), jnp.int32),
            pltpu.VMEM((CPT, CHUNK), jnp.int32),
            pltpu.VMEM((CHUNK, w), jnp.float32),
            pltpu.VMEM_SHARED((NP, w), jnp.float32),
            pltpu.SemaphoreType.DMA,
        ],
        compiler_params=pltpu.CompilerParams(use_tc_tiling_on_sc=False),
    )
    def agg(v_hbm, srcw_hbm, dstw_hbm, out_hbm, src_v, dst_v, buf, acc_sh, sem):
        c = lax.axis_index("c")
        s = lax.axis_index("s")
        wid = c * NS + s

        def zbody(i, carry):
            for hh in range(w // 16):
                buf[i, hh * 16:(hh + 1) * 16] = jnp.zeros((16,), jnp.float32)
            return carry

        lax.fori_loop(0, CHUNK, zbody, 0)
        row0 = s * RPT
        off = 0
        for sz in (128, 128, 128, 128, 120):
            pltpu.sync_copy(buf.at[pl.ds(0, sz)],
                            acc_sh.at[pl.ds(row0 + off, sz)])
            off += sz

        pltpu.sync_copy(srcw_hbm.at[wid], src_v)
        pltpu.sync_copy(dstw_hbm.at[wid], dst_v)
        plsc.subcore_barrier()

        def body(j, carry):
            pltpu.async_copy(v_hbm.at[src_v.at[j]], buf, sem).wait()
            pltpu.sync_copy(buf, acc_sh.at[dst_v.at[j]], add=True)
            return carry

        lax.fori_loop(0, CPT, body, 0)
        plsc.subcore_barrier()
        pltpu.sync_copy(acc_sh.at[pl.ds(row0, RPT)],
                        out_hbm.at[c, pl.ds(row0, RPT)])

    return agg


_agg32 = _make_agg(H)
_agg96 = _make_agg(3 * H)


def _xproj_kernel(x_ref, w_ref, o_ref):
    o_ref[0] = jnp.dot(x_ref[0], w_ref[...],
                       preferred_element_type=jnp.float32)


def _xterm1_kernel(xp_ref, ag_ref, o_ref):
    o_ref[0] = xp_ref[0] + ag_ref[0, 0] + ag_ref[0, 1]


def _xterm2_kernel(o1_ref, ag_ref, w_ref, o_ref):
    xa = o1_ref[0] + ag_ref[0, 0] + ag_ref[0, 1]
    o_ref[0] = jnp.dot(xa, w_ref[...], preferred_element_type=jnp.float32)


def _gate_a_kernel(xterm_ref, h_ref, aggh_ref, whzr_ref, bzr_ref,
                   z_ref, g_ref, xc_ref):
    h = h_ref[...]
    ha = h + aggh_ref[0] + aggh_ref[1]
    hz = jnp.dot(ha, whzr_ref[...],
                 preferred_element_type=jnp.float32) + bzr_ref[...]
    xt = xterm_ref[...]
    z = jax.nn.sigmoid(xt[:, :H] + hz[:, :H])
    r = jax.nn.sigmoid(xt[:, H:2 * H] + hz[:, H:])
    z_ref[...] = z
    g_ref[...] = r * h
    xc_ref[...] = xt[:, 2 * H:]


def _gate_b_kernel(z_ref, h_ref, g_ref, xc_ref, aggg_ref, whh_ref, bh_ref,
                   hn_ref):
    ga = g_ref[...] + aggg_ref[0] + aggg_ref[1]
    c = jnp.tanh(xc_ref[...] +
                 jnp.dot(ga, whh_ref[...], preferred_element_type=jnp.float32)
                 + bh_ref[...])
    z = z_ref[...]
    hn_ref[...] = z * h_ref[...] + (1.0 - z) * c


def _head_kernel(h_ref, w_ref, b_ref, o_ref):
    o_ref[...] = jax.nn.relu(
        jnp.dot(h_ref[...], w_ref[...], preferred_element_type=jnp.float32)
        + b_ref[...])


def _tc(body, out_shape, grid=None, in_specs=None, out_specs=None):
    kwargs = {}
    if grid is not None:
        kwargs = dict(grid=grid, in_specs=in_specs, out_specs=out_specs)
    return pl.pallas_call(body, out_shape=out_shape, **kwargs)


def kernel(x, edge_index, params):
    p1, p2 = params['l1'], params['l2']

    Wx1 = jnp.concatenate([p1['Wxz'], p1['Wxr'], p1['Wxh']], axis=1)
    Wx2 = jnp.concatenate([p2['Wxz'], p2['Wxr'], p2['Wxh']], axis=1)
    Whzr1 = jnp.concatenate([p1['Whz'], p1['Whr']], axis=1)
    Whzr2 = jnp.concatenate([p2['Whz'], p2['Whr']], axis=1)
    bzr1 = jnp.concatenate([p1['bz'], p1['br']]).reshape(1, 2 * H)
    bzr2 = jnp.concatenate([p2['bz'], p2['br']]).reshape(1, 2 * H)
    bh1 = p1['bh'].reshape(1, H)
    bh2 = p2['bh'].reshape(1, H)
    bfc = params['bfc'].reshape(1, P * OUT)

    pad = EP - E
    srcp = jnp.concatenate([edge_index[0],
                            jnp.zeros((pad,), jnp.int32)])
    dstp = jnp.concatenate([edge_index[1],
                            jnp.full((pad,), NP - 1, jnp.int32)])
    srcw = srcp.reshape(NW, CPT, CHUNK)
    dstw = dstp.reshape(NW, CPT, CHUNK)

    x_pad = jnp.pad(x, ((0, 0), (0, NP - N), (0, 0)))

    xp_all = _tc(
        _xproj_kernel,
        jax.ShapeDtypeStruct((T, NP, 3 * H), jnp.float32),
        grid=(T,),
        in_specs=[pl.BlockSpec((1, NP, D), lambda t: (t, 0, 0)),
                  pl.BlockSpec((D, 3 * H), lambda t: (0, 0))],
        out_specs=pl.BlockSpec((1, NP, 3 * H), lambda t: (t, 0, 0)),
    )(x_pad, Wx1)

    aggxp = lax.map(lambda v: _agg96(v, srcw, dstw), xp_all)

    xterm1 = _tc(
        _xterm1_kernel,
        jax.ShapeDtypeStruct((T, NP, 3 * H), jnp.float32),
        grid=(T,),
        in_specs=[pl.BlockSpec((1, NP, 3 * H), lambda t: (t, 0, 0)),
                  pl.BlockSpec((1, NC, NP, 3 * H), lambda t: (t, 0, 0, 0))],
        out_specs=pl.BlockSpec((1, NP, 3 * H), lambda t: (t, 0, 0)),
    )(xp_all, aggxp)

    def make_step(Whzr, bzr, Whh, bh, collect):
        def step(h, xterm_t):
            aggh = _agg32(h, srcw, dstw)
            z, g, xc = _tc(
                _gate_a_kernel,
                (jax.ShapeDtypeStruct((NP, H), jnp.float32),
                 jax.ShapeDtypeStruct((NP, H), jnp.float32),
                 jax.ShapeDtypeStruct((NP, H), jnp.float32)),
            )(xterm_t, h, aggh, Whzr, bzr)
            aggg = _agg32(g, srcw, dstw)
            hn = _tc(
                _gate_b_kernel,
                jax.ShapeDtypeStruct((NP, H), jnp.float32),
            )(z, h, g, xc, aggg, Whh, bh)
            return hn, (hn if collect else None)
        return step

    h0 = jnp.zeros((NP, H), jnp.float32)
    h1, out1 = lax.scan(make_step(Whzr1, bzr1, p1['Whh'], bh1, True),
                        h0, xterm1)

    aggo = lax.map(lambda v: _agg32(v, srcw, dstw), out1)
    xterm2 = _tc(
        _xterm2_kernel,
        jax.ShapeDtypeStruct((T, NP, 3 * H), jnp.float32),
        grid=(T,),
        in_specs=[pl.BlockSpec((1, NP, H), lambda t: (t, 0, 0)),
                  pl.BlockSpec((1, NC, NP, H), lambda t: (t, 0, 0, 0)),
                  pl.BlockSpec((H, 3 * H), lambda t: (0, 0))],
        out_specs=pl.BlockSpec((1, NP, 3 * H), lambda t: (t, 0, 0)),
    )(out1, aggo, Wx2)

    h2, _ = lax.scan(make_step(Whzr2, bzr2, p2['Whh'], bh2, False),
                     h1, xterm2)

    y = _tc(
        _head_kernel,
        jax.ShapeDtypeStruct((NP, P * OUT), jnp.float32),
    )(h2, params['Wfc'], bfc)

    y = y[:N].reshape(N, P, OUT)
    return jnp.swapaxes(y, 0, 1)
